# Initial kernel scaffold; baseline (speedup 1.0000x reference)
import jax
import jax.numpy as jnp
from jax.experimental import pallas as pl


def kernel(combined_output, weights):
    raise NotImplementedError("write your pallas kernel here")


# speedup vs baseline: 1.9930x; 1.9930x over previous
import jax
import jax.numpy as jnp
from jax.experimental import pallas as pl
from jax.experimental.pallas import tpu as pltpu

_ROWS_PER_BLOCK = 256


def _combine_body(x_ref, w_ref, o_ref):
    x = x_ref[...]
    w = w_ref[...]
    o_ref[...] = x[:, 0, :] * w[:, 0:1] + x[:, 1, :] * w[:, 1:2]


def kernel(combined_output, weights):
    B, T, K, D = combined_output.shape
    N = B * T
    x = combined_output.reshape(N, K, D)
    w = weights.reshape(N, K)
    R = _ROWS_PER_BLOCK
    grid = (N // R,)
    out = pl.pallas_call(
        _combine_body,
        grid=grid,
        in_specs=[
            pl.BlockSpec((R, K, D), lambda i: (i, 0, 0)),
            pl.BlockSpec((R, K), lambda i: (i, 0)),
        ],
        out_specs=pl.BlockSpec((R, D), lambda i: (i, 0)),
        out_shape=jax.ShapeDtypeStruct((N, D), combined_output.dtype),
    )(x, w)
    return out.reshape(B, T, D)
